# Initial kernel scaffold; baseline (speedup 1.0000x reference)
import jax
import jax.numpy as jnp
from jax.experimental import pallas as pl


def kernel(adj_indices, adj_values, user_emb, item_emb):
    raise NotImplementedError("write your pallas kernel here")


# speedup vs baseline: 3.9044x; 3.9044x over previous
import functools

import jax
import jax.numpy as jnp
from jax import lax
from jax.experimental import pallas as pl
from jax.experimental.pallas import tpu as pltpu
from jax.experimental.pallas import tpu_sc as plsc

NUM_USERS = 25000
NUM_ITEMS = 75000
NUM_LAYERS = 3
D = 32
B = 128
NCORES = 2
NSUB = 16


def _layer_body(nb, h, h16, pt, ego_hbm, src_hbm, dst_hbm, val_hbm, zeros_hbm,
                out_hbm, src_buf, dst_buf, val_buf, idx_buf, scale_buf,
                rows_buf, acc, sem):
    c = lax.axis_index("c")
    s = lax.axis_index("s")
    base_row = c * h

    pltpu.sync_copy(zeros_hbm, acc.at[pl.ds(s * h16, h16)])
    plsc.subcore_barrier()

    tile_e0 = s * pt

    def block_body(b, carry):
        e0 = tile_e0 + b * B
        pltpu.sync_copy(src_hbm.at[pl.ds(e0, B)], src_buf)
        pltpu.sync_copy(dst_hbm.at[pl.ds(e0, B)], dst_buf)
        pltpu.sync_copy(val_hbm.at[pl.ds(e0, B)], val_buf)
        pltpu.async_copy(ego_hbm.at[src_buf], rows_buf, sem).wait()

        for j in range(B // 16):
            dv = dst_buf[pl.ds(j * 16, 16)]
            lv = dv - base_row
            ok = (lv >= 0) & (lv < h)
            sc = jnp.where(ok, val_buf[pl.ds(j * 16, 16)], 0.0)
            lc = jnp.minimum(jnp.maximum(lv, 0), h - 1)
            idx_buf[pl.ds(j * 16, 16)] = lc
            scale_buf[pl.ds(j * 16, 16)] = sc

        def row_body(i, carry2):
            scv = scale_buf[pl.ds(i * 16, 16)]
            for u in range(16):
                e = i * 16 + u
                sc = scv[u]
                rows_buf[e, pl.ds(0, 16)] = rows_buf[e, pl.ds(0, 16)] * sc
                rows_buf[e, pl.ds(16, 16)] = rows_buf[e, pl.ds(16, 16)] * sc
            return carry2

        lax.fori_loop(0, B // 16, row_body, 0)

        pltpu.sync_copy(rows_buf, acc.at[idx_buf], add=True)
        return carry

    lax.fori_loop(0, nb, block_body, 0)
    plsc.subcore_barrier()

    pltpu.sync_copy(acc.at[pl.ds(s * h16, h16)],
                    out_hbm.at[pl.ds(base_row + s * h16, h16)])


def _make_layer(n, e_pad):
    h = n // NCORES
    h16 = h // NSUB
    pt = e_pad // NSUB
    nb = pt // B
    mesh = plsc.VectorSubcoreMesh(core_axis_name="c", subcore_axis_name="s")
    return pl.kernel(
        functools.partial(_layer_body, nb, h, h16, pt),
        out_type=jax.ShapeDtypeStruct((n, D), jnp.float32),
        mesh=mesh,
        scratch_types=[
            pltpu.VMEM((B,), jnp.int32),
            pltpu.VMEM((B,), jnp.int32),
            pltpu.VMEM((B,), jnp.float32),
            pltpu.VMEM((B,), jnp.int32),
            pltpu.VMEM((B,), jnp.float32),
            pltpu.VMEM((B, D), jnp.float32),
            pltpu.VMEM_SHARED((h, D), jnp.float32),
            pltpu.SemaphoreType.DMA,
        ],
        compiler_params=pltpu.CompilerParams(use_tc_tiling_on_sc=False),
        name="lightgcn_spmm_layer",
    )


def _mean_body(e0, e1, e2, e3, out):
    out[...] = (e0[...] + e1[...] + e2[...] + e3[...]) * 0.25


def _mean4(egos, n):
    rows = n * D // 128
    blk = 1000
    grid = rows // blk
    flat = [e.reshape(rows, 128) for e in egos]
    spec = pl.BlockSpec((blk, 128), lambda i: (i, 0))
    out = pl.pallas_call(
        _mean_body,
        out_shape=jax.ShapeDtypeStruct((rows, 128), jnp.float32),
        grid=(grid,),
        in_specs=[spec] * 4,
        out_specs=spec,
    )(*flat)
    return out.reshape(n, D)


def kernel(adj_indices, adj_values, user_emb, item_emb):
    n = user_emb.shape[0] + item_emb.shape[0]
    row_chunk = NCORES * NSUB * 8
    n_pad = ((n + row_chunk - 1) // row_chunk) * row_chunk
    e = adj_values.shape[0]
    chunk = NSUB * B
    e_pad = ((e + chunk - 1) // chunk) * chunk

    dst = adj_indices[0]
    src = adj_indices[1]
    pad = e_pad - e
    if pad:
        dst = jnp.pad(dst, (0, pad))
        src = jnp.pad(src, (0, pad))
        val = jnp.pad(adj_values, (0, pad))
    else:
        val = adj_values
    zeros = jnp.zeros((n_pad // NCORES // NSUB, D), jnp.float32)

    ego0 = jnp.concatenate(
        [user_emb, item_emb,
         jnp.zeros((n_pad - n, D), jnp.float32)], axis=0)
    layer = _make_layer(n_pad, e_pad)
    ego1 = layer(ego0, src, dst, val, zeros)
    ego2 = layer(ego1, src, dst, val, zeros)
    ego3 = layer(ego2, src, dst, val, zeros)

    final = _mean4([x[:n] for x in (ego0, ego1, ego2, ego3)], n)
    nu = user_emb.shape[0]
    return (final[:nu], final[nu:])


# speedup vs baseline: 7.1733x; 1.8372x over previous
import functools

import jax
import jax.numpy as jnp
from jax import lax
from jax.experimental import pallas as pl
from jax.experimental.pallas import tpu as pltpu
from jax.experimental.pallas import tpu_sc as plsc

NUM_USERS = 25000
NUM_ITEMS = 75000
NUM_LAYERS = 3
D = 32
B = 128
NCORES = 2
NSUB = 16


G = 16
CB = G * B


def _layer_body(nb, h, h16, pt, ego_hbm, src_hbm, dst_hbm, val_hbm, zeros_hbm,
                out_hbm, srcc, dstc, valc, idxc, scalec,
                rows, idxb, acc, semg, sems):
    c = lax.axis_index("c")
    s = lax.axis_index("s")
    base_row = c * h

    pltpu.sync_copy(zeros_hbm, acc.at[pl.ds(s * h16, h16)])
    plsc.subcore_barrier()

    tile_e0 = s * pt
    nchunk = nb // G

    def gather_start(j, slot):
        pltpu.async_copy(ego_hbm.at[srcc.at[pl.ds(j * B, B)]],
                         rows[slot], semg[slot])

    def gather_wait(j, slot):
        pltpu.make_async_copy(ego_hbm.at[srcc.at[pl.ds(j * B, B)]],
                              rows[slot], semg[slot]).wait()

    def scatter_start(slot):
        pltpu.async_copy(rows[slot], acc.at[idxb[slot]], sems[slot],
                         add=True)

    def scatter_wait(slot):
        pltpu.make_async_copy(rows[slot], acc.at[idxb[slot]],
                              sems[slot]).wait()

    def compute_block(j, slot):
        for k in range(B // 16):
            idxb[slot][pl.ds(k * 16, 16)] = idxc[pl.ds(j * B + k * 16, 16)]

        def row_body(i, carry2):
            scv = scalec[pl.ds(j * B + i * 16, 16)]
            for u in range(16):
                e = i * 16 + u
                sc = scv[u]
                rows[slot][e, pl.ds(0, 16)] = rows[slot][e, pl.ds(0, 16)] * sc
                rows[slot][e, pl.ds(16, 16)] = rows[slot][e, pl.ds(16, 16)] * sc
            return carry2

        lax.fori_loop(0, B // 16, row_body, 0)

    def chunk_body(ci, carry):
        e0 = tile_e0 + ci * CB
        pltpu.sync_copy(src_hbm.at[pl.ds(e0, CB)], srcc)
        pltpu.sync_copy(dst_hbm.at[pl.ds(e0, CB)], dstc)
        pltpu.sync_copy(val_hbm.at[pl.ds(e0, CB)], valc)

        def scale_body(i, carry2):
            dv = dstc[pl.ds(i * 16, 16)]
            lv = dv - base_row
            ok = (lv >= 0) & (lv < h)
            sc = jnp.where(ok, valc[pl.ds(i * 16, 16)], 0.0)
            lc = jnp.minimum(jnp.maximum(lv, 0), h - 1)
            idxc[pl.ds(i * 16, 16)] = lc
            scalec[pl.ds(i * 16, 16)] = sc
            return carry2

        lax.fori_loop(0, CB // 16, scale_body, 0)

        gather_start(0, 0)
        gather_start(1, 1)
        gather_wait(0, 0)
        compute_block(0, 0)
        scatter_start(0)
        scatter_wait(0)
        gather_start(2, 0)
        gather_wait(1, 1)
        compute_block(1, 1)
        scatter_start(1)

        def pair_body(p, carry2):
            j = 2 * p
            for slot in (0, 1):
                other = 1 - slot
                scatter_wait(other)
                gather_start(j + slot + 1, other)
                gather_wait(j + slot, slot)
                compute_block(j + slot, slot)
                scatter_start(slot)
            return carry2

        lax.fori_loop(1, G // 2 - 1, pair_body, 0)

        j = G - 2
        scatter_wait(1)
        gather_start(j + 1, 1)
        gather_wait(j, 0)
        compute_block(j, 0)
        scatter_start(0)
        gather_wait(j + 1, 1)
        compute_block(j + 1, 1)
        scatter_start(1)
        scatter_wait(0)
        scatter_wait(1)
        return carry

    lax.fori_loop(0, nchunk, chunk_body, 0)
    plsc.subcore_barrier()

    pltpu.sync_copy(acc.at[pl.ds(s * h16, h16)],
                    out_hbm.at[pl.ds(base_row + s * h16, h16)])


def _make_layer(n, e_pad):
    h = n // NCORES
    h16 = h // NSUB
    pt = e_pad // NSUB
    nb = pt // B
    mesh = plsc.VectorSubcoreMesh(core_axis_name="c", subcore_axis_name="s")

    def body(ego_hbm, src_hbm, dst_hbm, val_hbm, zeros_hbm, out_hbm,
             srcc, dstc, valc, idxc, scalec,
             rows0, rows1, idxb0, idxb1, acc,
             semg0, semg1, sems0, sems1):
        _layer_body(nb, h, h16, pt, ego_hbm, src_hbm, dst_hbm, val_hbm,
                    zeros_hbm, out_hbm, srcc, dstc, valc, idxc, scalec,
                    (rows0, rows1), (idxb0, idxb1),
                    acc, (semg0, semg1), (sems0, sems1))

    return pl.kernel(
        body,
        out_type=jax.ShapeDtypeStruct((n, D), jnp.float32),
        mesh=mesh,
        scratch_types=[
            pltpu.VMEM((CB,), jnp.int32),
            pltpu.VMEM((CB,), jnp.int32),
            pltpu.VMEM((CB,), jnp.float32),
            pltpu.VMEM((CB,), jnp.int32),
            pltpu.VMEM((CB,), jnp.float32),
            pltpu.VMEM((B, D), jnp.float32),
            pltpu.VMEM((B, D), jnp.float32),
            pltpu.VMEM((B,), jnp.int32),
            pltpu.VMEM((B,), jnp.int32),
            pltpu.VMEM_SHARED((h, D), jnp.float32),
            pltpu.SemaphoreType.DMA,
            pltpu.SemaphoreType.DMA,
            pltpu.SemaphoreType.DMA,
            pltpu.SemaphoreType.DMA,
        ],
        compiler_params=pltpu.CompilerParams(use_tc_tiling_on_sc=False),
        name="lightgcn_spmm_layer",
    )


def _mean_body(e0, e1, e2, e3, out):
    out[...] = (e0[...] + e1[...] + e2[...] + e3[...]) * 0.25


def _mean4(egos, n):
    rows = n * D // 128
    blk = 1000
    grid = rows // blk
    flat = [e.reshape(rows, 128) for e in egos]
    spec = pl.BlockSpec((blk, 128), lambda i: (i, 0))
    out = pl.pallas_call(
        _mean_body,
        out_shape=jax.ShapeDtypeStruct((rows, 128), jnp.float32),
        grid=(grid,),
        in_specs=[spec] * 4,
        out_specs=spec,
    )(*flat)
    return out.reshape(n, D)


def kernel(adj_indices, adj_values, user_emb, item_emb):
    n = user_emb.shape[0] + item_emb.shape[0]
    row_chunk = NCORES * NSUB * 8
    n_pad = ((n + row_chunk - 1) // row_chunk) * row_chunk
    e = adj_values.shape[0]
    chunk = NSUB * B * G
    e_pad = ((e + chunk - 1) // chunk) * chunk

    dst = adj_indices[0]
    src = adj_indices[1]
    pad = e_pad - e
    if pad:
        dst = jnp.pad(dst, (0, pad))
        src = jnp.pad(src, (0, pad))
        val = jnp.pad(adj_values, (0, pad))
    else:
        val = adj_values
    zeros = jnp.zeros((n_pad // NCORES // NSUB, D), jnp.float32)

    ego0 = jnp.concatenate(
        [user_emb, item_emb,
         jnp.zeros((n_pad - n, D), jnp.float32)], axis=0)
    layer = _make_layer(n_pad, e_pad)
    ego1 = layer(ego0, src, dst, val, zeros)
    ego2 = layer(ego1, src, dst, val, zeros)
    ego3 = layer(ego2, src, dst, val, zeros)

    final = _mean4([x[:n] for x in (ego0, ego1, ego2, ego3)], n)
    nu = user_emb.shape[0]
    return (final[:nu], final[nu:])
